# Initial kernel scaffold; baseline (speedup 1.0000x reference)
import jax
import jax.numpy as jnp
from jax.experimental import pallas as pl


def kernel(x, real_positions, wih_packed, whh_packed, bias_packed):
    raise NotImplementedError("write your pallas kernel here")


# speedup vs baseline: 7.5090x; 7.5090x over previous
import functools

import jax
import jax.numpy as jnp
from jax import lax
from jax.experimental import pallas as pl
from jax.experimental.pallas import tpu as pltpu

_ROWS = 256
_TCHUNK = 16


def _ceil_to(n, m):
    return ((n + m - 1) // m) * m


def _lstm_tile(places_ref, x_ref, wih_ref, whh_ref, b_ref, out_ref, gin_ref,
               *, hidden):
    S, Bt, D = x_ref.shape
    H = hidden
    C = gin_ref.shape[0] // Bt

    whh = whh_ref[...]
    bias = b_ref[...]
    places = places_ref[...]

    h = jnp.zeros((Bt, H), jnp.float32)
    c = jnp.zeros((Bt, H), jnp.float32)
    out = jnp.zeros((Bt, H), jnp.float32)

    def step(j, carry, base):
        h, c, out = carry
        r = pl.multiple_of(j * Bt, Bt)
        gates = (jnp.dot(h, whh, preferred_element_type=jnp.float32)
                 + gin_ref[pl.ds(r, Bt), :])
        i_g = jax.nn.sigmoid(gates[:, 0 * H:1 * H])
        f_g = jax.nn.sigmoid(gates[:, 1 * H:2 * H])
        o_g = jax.nn.sigmoid(gates[:, 2 * H:3 * H])
        g_g = jnp.tanh(gates[:, 3 * H:4 * H])
        c = f_g * c + i_g * g_g
        h = o_g * jnp.tanh(c)
        out = jnp.where(places == base + j, h, out)
        return h, c, out

    for k in range(S // C):
        xc = x_ref[k * C:(k + 1) * C].reshape(C * Bt, D)
        gin_ref[...] = (
            jnp.dot(xc, wih_ref[...], preferred_element_type=jnp.float32)
            + bias)
        h, c, out = lax.fori_loop(
            0, C, functools.partial(step, base=k * C), (h, c, out),
            unroll=4)

    out_ref[...] = out


@jax.jit
def kernel(x, real_positions, wih_packed, whh_packed, bias_packed):
    B, S, D = x.shape
    H, Gp = whh_packed.shape
    Bt = _ROWS
    Bp = _ceil_to(B, Bt)
    C = _TCHUNK if S % _TCHUNK == 0 else S

    x_tm = jnp.transpose(x.astype(jnp.bfloat16), (1, 0, 2))
    if Bp != B:
        x_tm = jnp.pad(x_tm, ((0, 0), (0, Bp - B), (0, 0)))

    lengths = jnp.sum(real_positions.astype(jnp.float32), axis=-1)
    places = lengths.astype(jnp.int32) - 1
    places = jnp.where(places < 0, places + S, places)[:, None]
    if Bp != B:
        places = jnp.pad(places, ((0, Bp - B), (0, 0)))

    out = pl.pallas_call(
        functools.partial(_lstm_tile, hidden=H),
        out_shape=jax.ShapeDtypeStruct((Bp, H), jnp.float32),
        grid_spec=pltpu.PrefetchScalarGridSpec(
            num_scalar_prefetch=0,
            grid=(Bp // Bt,),
            in_specs=[
                pl.BlockSpec((Bt, 1), lambda g: (g, 0)),
                pl.BlockSpec((S, Bt, D), lambda g: (0, g, 0)),
                pl.BlockSpec((D, Gp), lambda g: (0, 0)),
                pl.BlockSpec((H, Gp), lambda g: (0, 0)),
                pl.BlockSpec((1, Gp), lambda g: (0, 0)),
            ],
            out_specs=pl.BlockSpec((Bt, H), lambda g: (g, 0)),
            scratch_shapes=[pltpu.VMEM((C * Bt, Gp), jnp.float32)],
        ),
        compiler_params=pltpu.CompilerParams(
            dimension_semantics=("parallel",)),
    )(places, x_tm, wih_packed, whh_packed, bias_packed)

    return out[:B]


# speedup vs baseline: 8.5753x; 1.1420x over previous
import functools

import jax
import jax.numpy as jnp
from jax import lax
from jax.experimental import pallas as pl
from jax.experimental.pallas import tpu as pltpu

_ROWS = 256
_TCHUNK = 16


def _ceil_to(n, m):
    return ((n + m - 1) // m) * m


def _lstm_tile(places_ref, x_ref, wih_ref, whh_ref, b_ref, out_ref, gin_ref,
               *, hidden):
    S, Bt, D = x_ref.shape
    H = hidden
    C = gin_ref.shape[0] // Bt

    whh = whh_ref[...]
    bias = b_ref[...]
    places = places_ref[...]

    h = jnp.zeros((Bt, H), jnp.float32)
    c = jnp.zeros((Bt, H), jnp.float32)
    out = jnp.zeros((Bt, H), jnp.float32)

    def step(j, carry, base):
        h, c, out = carry
        r = pl.multiple_of(j * Bt, Bt)
        gates = (jnp.dot(h, whh, preferred_element_type=jnp.float32)
                 + gin_ref[pl.ds(r, Bt), :])
        i_g = jax.nn.sigmoid(gates[:, 0 * H:1 * H])
        f_g = jax.nn.sigmoid(gates[:, 1 * H:2 * H])
        o_g = jax.nn.sigmoid(gates[:, 2 * H:3 * H])
        g_g = jnp.tanh(gates[:, 3 * H:4 * H])
        c = f_g * c + i_g * g_g
        h = o_g * jnp.tanh(c)
        out = jnp.where(places == base + j, h, out)
        return h, c, out

    for k in range(S // C):
        xc = x_ref[k * C:(k + 1) * C].reshape(C * Bt, D)
        gin_ref[...] = (
            jnp.dot(xc, wih_ref[...], preferred_element_type=jnp.float32)
            + bias)
        for j in range(C):
            h, c, out = step(j, (h, c, out), base=k * C)

    out_ref[...] = out


@jax.jit
def kernel(x, real_positions, wih_packed, whh_packed, bias_packed):
    B, S, D = x.shape
    H, Gp = whh_packed.shape
    Bt = _ROWS
    Bp = _ceil_to(B, Bt)
    C = _TCHUNK if S % _TCHUNK == 0 else S

    x_tm = jnp.transpose(x.astype(jnp.bfloat16), (1, 0, 2))
    if Bp != B:
        x_tm = jnp.pad(x_tm, ((0, 0), (0, Bp - B), (0, 0)))

    lengths = jnp.sum(real_positions.astype(jnp.float32), axis=-1)
    places = lengths.astype(jnp.int32) - 1
    places = jnp.where(places < 0, places + S, places)[:, None]
    if Bp != B:
        places = jnp.pad(places, ((0, Bp - B), (0, 0)))

    out = pl.pallas_call(
        functools.partial(_lstm_tile, hidden=H),
        out_shape=jax.ShapeDtypeStruct((Bp, H), jnp.float32),
        grid_spec=pltpu.PrefetchScalarGridSpec(
            num_scalar_prefetch=0,
            grid=(Bp // Bt,),
            in_specs=[
                pl.BlockSpec((Bt, 1), lambda g: (g, 0)),
                pl.BlockSpec((S, Bt, D), lambda g: (0, g, 0)),
                pl.BlockSpec((D, Gp), lambda g: (0, 0)),
                pl.BlockSpec((H, Gp), lambda g: (0, 0)),
                pl.BlockSpec((1, Gp), lambda g: (0, 0)),
            ],
            out_specs=pl.BlockSpec((Bt, H), lambda g: (g, 0)),
            scratch_shapes=[pltpu.VMEM((C * Bt, Gp), jnp.float32)],
        ),
        compiler_params=pltpu.CompilerParams(
            dimension_semantics=("parallel",)),
    )(places, x_tm, wih_packed, whh_packed, bias_packed)

    return out[:B]
